# baseline (speedup 1.0000x reference)
import jax
import jax.numpy as jnp
from jax import lax
from jax.experimental import pallas as pl
from jax.experimental.pallas import tpu as pltpu
from jax.experimental.pallas import tpu_sc as plsc

_N = 10000
_D = 128
_H = 64
_CLS = 40
_K = 10

_NP = 10240
_EP = 360448
_RPT = _NP // 16
_EPT = _EP // 16
_BLK = 1024
_NBLK = _EPT // _BLK
_DROWS = _EP // 128
_PB = 160
_RB = 256
_NRB = _NP // _RB


def _sc_mesh():
    return plsc.VectorSubcoreMesh(core_axis_name="c", subcore_axis_name="s")


def _embed_body(xb, w1, b1r, h0, h1):
    h = jnp.maximum(jnp.dot(xb[...], w1[...],
                            preferred_element_type=jnp.float32) + b1r[...], 0.0)
    h0[...] = h[:, :32]
    h1[...] = h[:, 32:]


def _make_embed():
    f32 = jnp.float32
    o = jax.ShapeDtypeStruct
    return pl.pallas_call(
        _embed_body,
        grid=(_NRB,),
        in_specs=[
            pl.BlockSpec((_RB, _D), lambda i: (i, 0)),
            pl.BlockSpec((_D, _H), lambda i: (0, 0)),
            pl.BlockSpec((1, _H), lambda i: (0, 0)),
        ],
        out_specs=[
            pl.BlockSpec((_RB, 32), lambda i: (i, 0)),
            pl.BlockSpec((_RB, 32), lambda i: (i, 0)),
        ],
        out_shape=[o((_NP, 32), f32), o((_NP, 32), f32)],
    )


def _prop_body(src_h, dst2_h, h0_h, h1_h,
               uf0_h, uf1_h, d20_h, d21_h, hd0_h, hd1_h,
               agg_sh, sidx0, sidx1, didx0, didx1, didx2, didx3,
               rows0, rows1, aggc, d2c, hdc, zc,
               semg0, semg1, semi0, semi1, sems):
    c = lax.axis_index("c")
    s = lax.axis_index("s")
    row0 = pl.multiple_of(s * _RPT, 8)
    ebase = s * _EPT
    sidxs = (sidx0, sidx1)
    didxs = (didx0, didx1, didx2, didx3)
    rowss = (rows0, rows1)
    semgs = (semg0, semg1)
    semis = (semi0, semi1)

    def _psl(p):
        return pl.ds(pl.multiple_of(row0 + p * _PB, 8), _PB)

    def _eoff(b):
        return pl.multiple_of(ebase + b * _BLK, 128)

    def _doff(b):
        return pl.ds(pl.multiple_of(_eoff(b) // 128, 8), 8)

    def _idx_issue(b, with_src=True):
        semi = semis[b % 2]
        if with_src:
            pltpu.async_copy(src_h.at[pl.ds(_eoff(b), _BLK)], sidxs[b % 2],
                             semi)
        pltpu.async_copy(dst2_h.at[_doff(b)], didxs[b % 4], semi)

    def _idx_wait(b, with_src=True):
        semi = semis[b % 2]
        if with_src:
            pltpu.make_async_copy(
                src_h.at[pl.ds(0, _BLK)], sidxs[b % 2], semi).wait()
        pltpu.make_async_copy(dst2_h.at[pl.ds(0, 8)], didxs[b % 4], semi).wait()

    def _gather_issue(b):
        sidx, rows, semg = sidxs[b % 2], rowss[b % 2], semgs[b % 2]

        @pl.when(c == 0)
        def _():
            pltpu.async_copy(uf0_h.at[sidx], rows, semg)

        @pl.when(c == 1)
        def _():
            pltpu.async_copy(uf1_h.at[sidx], rows, semg)

    def _gather_wait(b):
        pltpu.make_async_copy(
            uf0_h.at[sidxs[b % 2]], rowss[b % 2], semgs[b % 2]).wait()

    def _scatter_issue(b, ones=False):
        didx = didxs[b % 4]
        for j in range(8):
            srcsl = rows0.at[pl.ds(0, 128)] if ones else (
                rowss[b % 2].at[pl.ds(j * 128, 128)])
            pltpu.async_copy(srcsl, agg_sh.at[didx.at[j]], sems, add=True)

    def _scatter_drain(b, ones=False):
        didx = didxs[b % 4]
        for j in range(8):
            srcsl = rows0.at[pl.ds(0, 128)] if ones else (
                rowss[b % 2].at[pl.ds(j * 128, 128)])
            pltpu.make_async_copy(srcsl, agg_sh.at[didx.at[j]], sems).wait()

    @pl.loop(0, _PB)
    def _(r):
        z16 = jnp.zeros((16,), jnp.float32)
        zc[r, pl.ds(0, 16)] = z16
        zc[r, pl.ds(16, 16)] = z16

    @pl.loop(0, 128)
    def _(r):
        o16 = jnp.ones((16,), jnp.float32)
        rows0[r, pl.ds(0, 16)] = o16
        rows0[r, pl.ds(16, 16)] = o16

    for p in range(_RPT // _PB):
        pltpu.sync_copy(zc, agg_sh.at[_psl(p)])

    plsc.subcore_barrier()

    pltpu.sync_copy(dst2_h.at[_doff(0)], didx0)
    _idx_issue(1, with_src=False)
    for b in range(_NBLK):
        if b >= 1:
            _scatter_drain(b - 1, ones=True)
        if b + 1 < _NBLK:
            _idx_wait(b + 1, with_src=False)
        _scatter_issue(b, ones=True)
        if b + 2 < _NBLK:
            _idx_issue(b + 2, with_src=False)
    _scatter_drain(_NBLK - 1, ones=True)

    plsc.subcore_barrier()

    for p in range(_RPT // _PB):
        psl = _psl(p)
        pltpu.sync_copy(agg_sh.at[psl], aggc)

        @pl.when(c == 0)
        def _():
            pltpu.sync_copy(h0_h.at[psl], hdc)

        @pl.when(c == 1)
        def _():
            pltpu.sync_copy(h1_h.at[psl], hdc)

        @pl.loop(0, _PB)
        def _(r):
            for cc in (0, 16):
                sl = pl.ds(cc, 16)
                deg = aggc[r, sl]
                xi = plsc.bitcast(deg, jnp.int32)
                yi = jnp.full((16,), 0x5F3759DF, jnp.int32) - (xi >> 1)
                y = plsc.bitcast(yi, jnp.float32)
                y = y * (1.5 - 0.5 * deg * y * y)
                y = y * (1.5 - 0.5 * deg * y * y)
                y = y * (1.5 - 0.5 * deg * y * y)
                dis = jnp.where(deg > 0.0, y, 0.0)
                u16 = dis * hdc[r, sl]
                aggc[r, sl] = u16
                d2c[r, sl] = 0.9 * dis * dis
                hdc[r, sl] = 0.1 * u16

        @pl.when(c == 0)
        def _():
            pltpu.sync_copy(aggc, uf0_h.at[psl])
            pltpu.sync_copy(d2c, d20_h.at[psl])
            pltpu.sync_copy(hdc, hd0_h.at[psl])

        @pl.when(c == 1)
        def _():
            pltpu.sync_copy(aggc, uf1_h.at[psl])
            pltpu.sync_copy(d2c, d21_h.at[psl])
            pltpu.sync_copy(hdc, hd1_h.at[psl])

        pltpu.sync_copy(zc, agg_sh.at[psl])

    plsc.subcore_barrier()

    @pl.loop(0, _K)
    def _(k):
        pltpu.sync_copy(src_h.at[pl.ds(_eoff(0), _BLK)], sidx0)
        pltpu.sync_copy(dst2_h.at[_doff(0)], didx0)
        _gather_issue(0)
        _idx_issue(1)
        for b in range(_NBLK):
            if b >= 1:
                _scatter_drain(b - 1)
            if b + 1 < _NBLK:
                _idx_wait(b + 1)
                _gather_issue(b + 1)
            _gather_wait(b)
            _scatter_issue(b)
            if b + 2 < _NBLK:
                _idx_issue(b + 2)
        _scatter_drain(_NBLK - 1)

        plsc.subcore_barrier()
        for p in range(_RPT // _PB):
            psl = _psl(p)
            pltpu.sync_copy(agg_sh.at[psl], aggc)

            @pl.when(c == 0)
            def _():
                pltpu.sync_copy(d20_h.at[psl], d2c)
                pltpu.sync_copy(hd0_h.at[psl], hdc)

            @pl.when(c == 1)
            def _():
                pltpu.sync_copy(d21_h.at[psl], d2c)
                pltpu.sync_copy(hd1_h.at[psl], hdc)

            @pl.loop(0, _PB)
            def _(r):
                for cc in (0, 16):
                    sl = pl.ds(cc, 16)
                    aggc[r, sl] = d2c[r, sl] * aggc[r, sl] + hdc[r, sl]

            @pl.when(c == 0)
            def _():
                pltpu.sync_copy(aggc, uf0_h.at[psl])

            @pl.when(c == 1)
            def _():
                pltpu.sync_copy(aggc, uf1_h.at[psl])

            pltpu.sync_copy(zc, agg_sh.at[psl])

        plsc.subcore_barrier()


def _make_prop():
    f32 = jnp.float32
    o = jax.ShapeDtypeStruct
    return pl.kernel(
        _prop_body,
        out_type=[o((_NP, 32), f32), o((_NP, 32), f32),
                  o((_NP, 32), f32), o((_NP, 32), f32),
                  o((_NP, 32), f32), o((_NP, 32), f32)],
        mesh=_sc_mesh(),
        compiler_params=pltpu.CompilerParams(use_tc_tiling_on_sc=False, needs_layout_passes=False),
        scratch_types=[
            pltpu.VMEM_SHARED((_NP, 32), f32),
            pltpu.VMEM((_BLK,), jnp.int32),
            pltpu.VMEM((_BLK,), jnp.int32),
            pltpu.VMEM((8, 128), jnp.int32),
            pltpu.VMEM((8, 128), jnp.int32),
            pltpu.VMEM((8, 128), jnp.int32),
            pltpu.VMEM((8, 128), jnp.int32),
            pltpu.VMEM((_BLK, 32), f32),
            pltpu.VMEM((_BLK, 32), f32),
            pltpu.VMEM((_PB, 32), f32),
            pltpu.VMEM((_PB, 32), f32),
            pltpu.VMEM((_PB, 32), f32),
            pltpu.VMEM((_PB, 32), f32),
            pltpu.SemaphoreType.DMA,
            pltpu.SemaphoreType.DMA,
            pltpu.SemaphoreType.DMA,
            pltpu.SemaphoreType.DMA,
            pltpu.SemaphoreType.DMA,
        ],
    )


def _final_body(u0b, u1b, d2b, w2, b2r, lsm, xo, sm):
    r = lax.rsqrt(jnp.maximum(d2b[...], 1e-30) * (1.0 / 0.9))
    z = jnp.concatenate([u0b[...] * r, u1b[...] * r], axis=1)
    logits = jnp.dot(z, w2[...], preferred_element_type=jnp.float32) + b2r[...]
    m = jnp.max(logits, axis=1, keepdims=True)
    ex = jnp.exp(logits - m)
    ssum = jnp.sum(ex, axis=1, keepdims=True)
    xo[...] = logits
    lsm[...] = logits - m - jnp.log(ssum)
    sm[...] = ex / ssum


def _make_final():
    f32 = jnp.float32
    o = jax.ShapeDtypeStruct
    return pl.pallas_call(
        _final_body,
        grid=(_NRB,),
        in_specs=[
            pl.BlockSpec((_RB, 32), lambda i: (i, 0)),
            pl.BlockSpec((_RB, 32), lambda i: (i, 0)),
            pl.BlockSpec((_RB, 32), lambda i: (i, 0)),
            pl.BlockSpec((_H, 128), lambda i: (0, 0)),
            pl.BlockSpec((1, 128), lambda i: (0, 0)),
        ],
        out_specs=[
            pl.BlockSpec((_RB, 128), lambda i: (i, 0)),
            pl.BlockSpec((_RB, 128), lambda i: (i, 0)),
            pl.BlockSpec((_RB, 128), lambda i: (i, 0)),
        ],
        out_shape=[o((_NP, 128), f32), o((_NP, 128), f32), o((_NP, 128), f32)],
    )


def kernel(x, edge_index, e_w, idx, W1, b1, W2, b2):
    del e_w, idx
    n_extra = _EP - (edge_index.shape[1] + _N)
    loops = jnp.arange(_N, dtype=jnp.int32)
    padv = _N + (jnp.arange(n_extra, dtype=jnp.int32) % (_NP - _N))
    src = jnp.concatenate([edge_index[0], loops, padv])
    dst = jnp.concatenate([edge_index[1], loops, padv])
    dst2 = dst.reshape(_DROWS, 128)

    xp = jnp.pad(x, ((0, _NP - _N), (0, 0)))
    b1r = b1.reshape(1, _H)
    w2p = jnp.pad(W2, ((0, 0), (0, 128 - _CLS)))
    b2r = jnp.concatenate(
        [b2, jnp.full((128 - _CLS,), -1e30, jnp.float32)]).reshape(1, 128)

    h0, h1 = _make_embed()(xp, W1, b1r)
    uf0, uf1, d20, _, _, _ = _make_prop()(src, dst2, h0, h1)
    lsm, xo, sm = _make_final()(uf0, uf1, d20, w2p, b2r)
    return (lsm[:_N, :_CLS], xo[:_N, :_CLS], 0.0, sm[:_N, :_CLS])


# speedup vs baseline: 1.0503x; 1.0503x over previous
import jax
import jax.numpy as jnp
from jax import lax
from jax.experimental import pallas as pl
from jax.experimental.pallas import tpu as pltpu
from jax.experimental.pallas import tpu_sc as plsc

_N = 10000
_D = 128
_H = 64
_CLS = 40
_K = 10

_NP = 10240
_EP = 327680
_RPT = _NP // 16
_EPT = _EP // 16
_BLK = 1024
_NBLK = _EPT // _BLK
_DROWS = _EP // 128
_PB = 160
_RB = 256
_NRB = _NP // _RB
_FRB = 400
_ERB = 512


def _sc_mesh():
    return plsc.VectorSubcoreMesh(core_axis_name="c", subcore_axis_name="s")


def _embed_body(xb, w1, b1r, h0, h1):
    h = jnp.maximum(jnp.dot(xb[...], w1[...],
                            preferred_element_type=jnp.float32) + b1r[...], 0.0)
    h0[...] = h[:, :32]
    h1[...] = h[:, 32:]


def _make_embed():
    f32 = jnp.float32
    o = jax.ShapeDtypeStruct
    return pl.pallas_call(
        _embed_body,
        grid=(_NP // _ERB,),
        in_specs=[
            pl.BlockSpec((_ERB, _D), lambda i: (i, 0)),
            pl.BlockSpec((_D, _H), lambda i: (0, 0)),
            pl.BlockSpec((1, _H), lambda i: (0, 0)),
        ],
        out_specs=[
            pl.BlockSpec((_ERB, 32), lambda i: (i, 0)),
            pl.BlockSpec((_ERB, 32), lambda i: (i, 0)),
        ],
        out_shape=[o((_NP, 32), f32), o((_NP, 32), f32)],
    )


def _prop_body(src_h, dst2_h, h0_h, h1_h,
               uf0_h, uf1_h, d20_h, d21_h, hd0_h, hd1_h,
               agg_sh, sidx0, sidx1, didx0, didx1, didx2, didx3,
               rows0, rows1, aggc, d2c, hdc, uc, zc,
               semg0, semg1, semi0, semi1, sems, semp):
    c = lax.axis_index("c")
    s = lax.axis_index("s")
    row0 = pl.multiple_of(s * _RPT, 8)
    ebase = s * _EPT
    sidxs = (sidx0, sidx1)
    didxs = (didx0, didx1, didx2, didx3)
    rowss = (rows0, rows1)
    semgs = (semg0, semg1)
    semis = (semi0, semi1)

    def _psl(p):
        return pl.ds(pl.multiple_of(row0 + p * _PB, 8), _PB)

    def _eoff(b):
        return pl.multiple_of(ebase + b * _BLK, 128)

    def _doff(b):
        return pl.ds(pl.multiple_of(_eoff(b) // 128, 8), 8)

    def _idx_issue(b, with_src=True):
        semi = semis[b % 2]
        if with_src:
            pltpu.async_copy(src_h.at[pl.ds(_eoff(b), _BLK)], sidxs[b % 2],
                             semi)
        pltpu.async_copy(dst2_h.at[_doff(b)], didxs[b % 4], semi)

    def _idx_wait(b, with_src=True):
        semi = semis[b % 2]
        if with_src:
            pltpu.make_async_copy(
                src_h.at[pl.ds(0, _BLK)], sidxs[b % 2], semi).wait()
        pltpu.make_async_copy(dst2_h.at[pl.ds(0, 8)], didxs[b % 4], semi).wait()

    def _gather_issue(b):
        sidx, rows, semg = sidxs[b % 2], rowss[b % 2], semgs[b % 2]

        @pl.when(c == 0)
        def _():
            pltpu.async_copy(uf0_h.at[sidx], rows, semg)

        @pl.when(c == 1)
        def _():
            pltpu.async_copy(uf1_h.at[sidx], rows, semg)

    def _gather_wait(b):
        pltpu.make_async_copy(
            uf0_h.at[sidxs[b % 2]], rowss[b % 2], semgs[b % 2]).wait()

    def _scatter_issue(b, ones=False):
        didx = didxs[b % 4]
        for j in range(8):
            srcsl = rows0.at[pl.ds(0, 128)] if ones else (
                rowss[b % 2].at[pl.ds(j * 128, 128)])
            pltpu.async_copy(srcsl, agg_sh.at[didx.at[j]], sems, add=True)

    def _scatter_drain(b, ones=False):
        didx = didxs[b % 4]
        for j in range(8):
            srcsl = rows0.at[pl.ds(0, 128)] if ones else (
                rowss[b % 2].at[pl.ds(j * 128, 128)])
            pltpu.make_async_copy(srcsl, agg_sh.at[didx.at[j]], sems).wait()

    @pl.loop(0, _PB)
    def _(r):
        z16 = jnp.zeros((16,), jnp.float32)
        zc[r, pl.ds(0, 16)] = z16
        zc[r, pl.ds(16, 16)] = z16

    @pl.loop(0, 128)
    def _(r):
        o16 = jnp.ones((16,), jnp.float32)
        rows0[r, pl.ds(0, 16)] = o16
        rows0[r, pl.ds(16, 16)] = o16

    for p in range(_RPT // _PB):
        pltpu.sync_copy(zc, agg_sh.at[_psl(p)])

    plsc.subcore_barrier()

    pltpu.sync_copy(dst2_h.at[_doff(0)], didx0)
    _idx_issue(1, with_src=False)
    for b in range(_NBLK):
        if b >= 1:
            _scatter_drain(b - 1, ones=True)
        if b + 1 < _NBLK:
            _idx_wait(b + 1, with_src=False)
        _scatter_issue(b, ones=True)
        if b + 2 < _NBLK:
            _idx_issue(b + 2, with_src=False)
    _scatter_drain(_NBLK - 1, ones=True)

    plsc.subcore_barrier()

    for p in range(_RPT // _PB):
        psl = _psl(p)
        pltpu.sync_copy(agg_sh.at[psl], aggc)

        @pl.when(c == 0)
        def _():
            pltpu.sync_copy(h0_h.at[psl], hdc)

        @pl.when(c == 1)
        def _():
            pltpu.sync_copy(h1_h.at[psl], hdc)

        @pl.loop(0, _PB)
        def _(r):
            for cc in (0, 16):
                sl = pl.ds(cc, 16)
                deg = aggc[r, sl] + 1.0
                xi = plsc.bitcast(deg, jnp.int32)
                yi = jnp.full((16,), 0x5F3759DF, jnp.int32) - (xi >> 1)
                y = plsc.bitcast(yi, jnp.float32)
                y = y * (1.5 - 0.5 * deg * y * y)
                y = y * (1.5 - 0.5 * deg * y * y)
                y = y * (1.5 - 0.5 * deg * y * y)
                dis = jnp.where(deg > 0.0, y, 0.0)
                u16 = dis * hdc[r, sl]
                aggc[r, sl] = u16
                d2c[r, sl] = 0.9 * dis * dis
                hdc[r, sl] = 0.1 * u16

        @pl.when(c == 0)
        def _():
            pltpu.sync_copy(aggc, uf0_h.at[psl])
            pltpu.sync_copy(d2c, d20_h.at[psl])
            pltpu.sync_copy(hdc, hd0_h.at[psl])

        @pl.when(c == 1)
        def _():
            pltpu.sync_copy(aggc, uf1_h.at[psl])
            pltpu.sync_copy(d2c, d21_h.at[psl])
            pltpu.sync_copy(hdc, hd1_h.at[psl])

        pltpu.sync_copy(zc, agg_sh.at[psl])

    plsc.subcore_barrier()

    @pl.loop(0, _K)
    def _(k):
        pltpu.sync_copy(src_h.at[pl.ds(_eoff(0), _BLK)], sidx0)
        pltpu.sync_copy(dst2_h.at[_doff(0)], didx0)
        _gather_issue(0)
        _idx_issue(1)
        for b in range(_NBLK):
            if b >= 1:
                _scatter_drain(b - 1)
            if b + 1 < _NBLK:
                _idx_wait(b + 1)
                _gather_issue(b + 1)
            _gather_wait(b)
            _scatter_issue(b)
            if b + 2 < _NBLK:
                _idx_issue(b + 2)
        _scatter_drain(_NBLK - 1)

        plsc.subcore_barrier()
        for p in range(_RPT // _PB):
            psl = _psl(p)
            pltpu.sync_copy(agg_sh.at[psl], aggc)

            @pl.when(c == 0)
            def _():
                pltpu.sync_copy(d20_h.at[psl], d2c)
                pltpu.sync_copy(hd0_h.at[psl], hdc)
                pltpu.sync_copy(uf0_h.at[psl], uc)

            @pl.when(c == 1)
            def _():
                pltpu.sync_copy(d21_h.at[psl], d2c)
                pltpu.sync_copy(hd1_h.at[psl], hdc)
                pltpu.sync_copy(uf1_h.at[psl], uc)

            @pl.loop(0, _PB)
            def _(r):
                for cc in (0, 16):
                    sl = pl.ds(cc, 16)
                    aggc[r, sl] = (d2c[r, sl] * (aggc[r, sl] + uc[r, sl])
                                   + hdc[r, sl])

            @pl.when(c == 0)
            def _():
                pltpu.sync_copy(aggc, uf0_h.at[psl])

            @pl.when(c == 1)
            def _():
                pltpu.sync_copy(aggc, uf1_h.at[psl])

            pltpu.sync_copy(zc, agg_sh.at[psl])

        plsc.subcore_barrier()


def _make_prop():
    f32 = jnp.float32
    o = jax.ShapeDtypeStruct
    return pl.kernel(
        _prop_body,
        out_type=[o((_NP, 32), f32), o((_NP, 32), f32),
                  o((_NP, 32), f32), o((_NP, 32), f32),
                  o((_NP, 32), f32), o((_NP, 32), f32)],
        mesh=_sc_mesh(),
        compiler_params=pltpu.CompilerParams(use_tc_tiling_on_sc=False, needs_layout_passes=False),
        scratch_types=[
            pltpu.VMEM_SHARED((_NP, 32), f32),
            pltpu.VMEM((_BLK,), jnp.int32),
            pltpu.VMEM((_BLK,), jnp.int32),
            pltpu.VMEM((8, 128), jnp.int32),
            pltpu.VMEM((8, 128), jnp.int32),
            pltpu.VMEM((8, 128), jnp.int32),
            pltpu.VMEM((8, 128), jnp.int32),
            pltpu.VMEM((_BLK, 32), f32),
            pltpu.VMEM((_BLK, 32), f32),
            pltpu.VMEM((_PB, 32), f32),
            pltpu.VMEM((_PB, 32), f32),
            pltpu.VMEM((_PB, 32), f32),
            pltpu.VMEM((_PB, 32), f32),
            pltpu.VMEM((_PB, 32), f32),
            pltpu.SemaphoreType.DMA,
            pltpu.SemaphoreType.DMA,
            pltpu.SemaphoreType.DMA,
            pltpu.SemaphoreType.DMA,
            pltpu.SemaphoreType.DMA,
            pltpu.SemaphoreType.DMA,
        ],
    )


def _final_body(u0b, u1b, d2b, w2, b2r, lsm, xo, sm):
    r = lax.rsqrt(jnp.maximum(d2b[...], 1e-30) * (1.0 / 0.9))
    z = jnp.concatenate([u0b[...] * r, u1b[...] * r], axis=1)
    logits = jnp.dot(z, w2[...], preferred_element_type=jnp.float32) + b2r[...]
    m = jnp.max(logits, axis=1, keepdims=True)
    ex = jnp.exp(logits - m)
    ssum = jnp.sum(ex, axis=1, keepdims=True)
    xo[...] = logits[:, :_CLS]
    lsm[...] = (logits - m - jnp.log(ssum))[:, :_CLS]
    sm[...] = (ex / ssum)[:, :_CLS]


def _make_final():
    f32 = jnp.float32
    o = jax.ShapeDtypeStruct
    return pl.pallas_call(
        _final_body,
        grid=(_N // _FRB,),
        in_specs=[
            pl.BlockSpec((_FRB, 32), lambda i: (i, 0)),
            pl.BlockSpec((_FRB, 32), lambda i: (i, 0)),
            pl.BlockSpec((_FRB, 32), lambda i: (i, 0)),
            pl.BlockSpec((_H, 128), lambda i: (0, 0)),
            pl.BlockSpec((1, 128), lambda i: (0, 0)),
        ],
        out_specs=[
            pl.BlockSpec((_FRB, _CLS), lambda i: (i, 0)),
            pl.BlockSpec((_FRB, _CLS), lambda i: (i, 0)),
            pl.BlockSpec((_FRB, _CLS), lambda i: (i, 0)),
        ],
        out_shape=[o((_N, _CLS), f32), o((_N, _CLS), f32), o((_N, _CLS), f32)],
    )


def kernel(x, edge_index, e_w, idx, W1, b1, W2, b2):
    del e_w, idx
    n_extra = _EP - edge_index.shape[1]
    padv = _N + (jnp.arange(n_extra, dtype=jnp.int32) % (_NP - _N))
    src = jnp.concatenate([edge_index[0], padv])
    dst = jnp.concatenate([edge_index[1], padv])
    dst2 = dst.reshape(_DROWS, 128)

    xp = jnp.pad(x, ((0, _NP - _N), (0, 0)))
    b1r = b1.reshape(1, _H)
    w2p = jnp.pad(W2, ((0, 0), (0, 128 - _CLS)))
    b2r = jnp.concatenate(
        [b2, jnp.full((128 - _CLS,), -1e30, jnp.float32)]).reshape(1, 128)

    h0, h1 = _make_embed()(xp, W1, b1r)
    uf0, uf1, d20, _, _, _ = _make_prop()(src, dst2, h0, h1)
    lsm, xo, sm = _make_final()(uf0, uf1, d20, w2p, b2r)
    return (lsm, xo, 0.0, sm)


# speedup vs baseline: 1.1170x; 1.0635x over previous
import jax
import jax.numpy as jnp
from jax import lax
from jax.experimental import pallas as pl
from jax.experimental.pallas import tpu as pltpu
from jax.experimental.pallas import tpu_sc as plsc

_N = 10000
_D = 128
_H = 64
_CLS = 40
_K = 10

_NP = 10240
_EP = 327680
_RPT = _NP // 16
_EPT = _EP // 16
_BLK = 1024
_NBLK = _EPT // _BLK
_DROWS = _EP // 128
_PB = 160
_RB = 256
_NRB = _NP // _RB
_FRB = 400
_ERB = 512


def _sc_mesh():
    return plsc.VectorSubcoreMesh(core_axis_name="c", subcore_axis_name="s")


def _embed_body(xb, w1, b1r, h0, h1):
    h = jnp.maximum(jnp.dot(xb[...], w1[...],
                            preferred_element_type=jnp.float32) + b1r[...], 0.0)
    h0[...] = h[:, :32]
    h1[...] = h[:, 32:]


def _make_embed():
    f32 = jnp.float32
    o = jax.ShapeDtypeStruct
    return pl.pallas_call(
        _embed_body,
        grid=(_NP // _ERB,),
        in_specs=[
            pl.BlockSpec((_ERB, _D), lambda i: (i, 0)),
            pl.BlockSpec((_D, _H), lambda i: (0, 0)),
            pl.BlockSpec((1, _H), lambda i: (0, 0)),
        ],
        out_specs=[
            pl.BlockSpec((_ERB, 32), lambda i: (i, 0)),
            pl.BlockSpec((_ERB, 32), lambda i: (i, 0)),
        ],
        out_shape=[o((_NP, 32), f32), o((_NP, 32), f32)],
    )


def _prop_body(src_h, dst2_h, h0_h, h1_h,
               uf0_h, uf1_h, d20_h, d21_h, hd0_h, hd1_h,
               agg_sh, sidx0, sidx1, didx0, didx1, didx2, didx3,
               rows0, rows1, aggc, d2c, hdc, uc, zc,
               semg0, semg1, semi0, semi1, sems, semp):
    c = lax.axis_index("c")
    s = lax.axis_index("s")
    row0 = pl.multiple_of(s * _RPT, 8)
    ebase = s * _EPT
    sidxs = (sidx0, sidx1)
    didxs = (didx0, didx1, didx2, didx3)
    rowss = (rows0, rows1)
    semgs = (semg0, semg1)
    semis = (semi0, semi1)

    def _psl(p):
        return pl.ds(pl.multiple_of(row0 + p * _PB, 8), _PB)

    def _eoff(b):
        return pl.multiple_of(ebase + b * _BLK, 128)

    def _doff(b):
        return pl.ds(pl.multiple_of(_eoff(b) // 128, 8), 8)

    def _idx_issue(b, with_src=True):
        semi = semis[b % 2]
        if with_src:
            pltpu.async_copy(src_h.at[pl.ds(_eoff(b), _BLK)], sidxs[b % 2],
                             semi)
        pltpu.async_copy(dst2_h.at[_doff(b)], didxs[b % 4], semi)

    def _idx_wait(b, with_src=True):
        semi = semis[b % 2]
        if with_src:
            pltpu.make_async_copy(
                src_h.at[pl.ds(0, _BLK)], sidxs[b % 2], semi).wait()
        pltpu.make_async_copy(dst2_h.at[pl.ds(0, 8)], didxs[b % 4], semi).wait()

    def _gather_issue(b):
        sidx, rows, semg = sidxs[b % 2], rowss[b % 2], semgs[b % 2]

        @pl.when(c == 0)
        def _():
            pltpu.async_copy(uf0_h.at[sidx], rows, semg)

        @pl.when(c == 1)
        def _():
            pltpu.async_copy(uf1_h.at[sidx], rows, semg)

    def _gather_wait(b):
        pltpu.make_async_copy(
            uf0_h.at[sidxs[b % 2]], rowss[b % 2], semgs[b % 2]).wait()

    def _scatter_issue(b, ones=False):
        didx = didxs[b % 4]
        for j in range(8):
            srcsl = rows0.at[pl.ds(0, 128)] if ones else (
                rowss[b % 2].at[pl.ds(j * 128, 128)])
            pltpu.async_copy(srcsl, agg_sh.at[didx.at[j]], sems, add=True)

    def _scatter_drain(b, ones=False):
        didx = didxs[b % 4]
        for j in range(8):
            srcsl = rows0.at[pl.ds(0, 128)] if ones else (
                rowss[b % 2].at[pl.ds(j * 128, 128)])
            pltpu.make_async_copy(srcsl, agg_sh.at[didx.at[j]], sems).wait()

    @pl.loop(0, _PB)
    def _(r):
        z16 = jnp.zeros((16,), jnp.float32)
        zc[r, pl.ds(0, 16)] = z16
        zc[r, pl.ds(16, 16)] = z16

    @pl.loop(0, 128)
    def _(r):
        o16 = jnp.ones((16,), jnp.float32)
        rows0[r, pl.ds(0, 16)] = o16
        rows0[r, pl.ds(16, 16)] = o16

    for p in range(_RPT // _PB):
        pltpu.sync_copy(zc, agg_sh.at[_psl(p)])

    plsc.subcore_barrier()

    pltpu.sync_copy(dst2_h.at[_doff(0)], didx0)
    _idx_issue(1, with_src=False)
    for b in range(_NBLK):
        if b >= 1:
            _scatter_drain(b - 1, ones=True)
        if b + 1 < _NBLK:
            _idx_wait(b + 1, with_src=False)
        _scatter_issue(b, ones=True)
        if b + 2 < _NBLK:
            _idx_issue(b + 2, with_src=False)
    _scatter_drain(_NBLK - 1, ones=True)

    plsc.subcore_barrier()

    for p in range(_RPT // _PB):
        psl = _psl(p)
        pltpu.sync_copy(agg_sh.at[psl], aggc)

        @pl.when(c == 0)
        def _():
            pltpu.sync_copy(h0_h.at[psl], hdc)

        @pl.when(c == 1)
        def _():
            pltpu.sync_copy(h1_h.at[psl], hdc)

        @pl.loop(0, _PB)
        def _(r):
            for cc in (0, 16):
                sl = pl.ds(cc, 16)
                deg = aggc[r, sl] + 1.0
                xi = plsc.bitcast(deg, jnp.int32)
                yi = jnp.full((16,), 0x5F3759DF, jnp.int32) - (xi >> 1)
                y = plsc.bitcast(yi, jnp.float32)
                y = y * (1.5 - 0.5 * deg * y * y)
                y = y * (1.5 - 0.5 * deg * y * y)
                y = y * (1.5 - 0.5 * deg * y * y)
                dis = jnp.where(deg > 0.0, y, 0.0)
                u16 = dis * hdc[r, sl]
                aggc[r, sl] = u16
                d2c[r, sl] = 0.9 * dis * dis
                hdc[r, sl] = 0.1 * u16

        @pl.when(c == 0)
        def _():
            pltpu.sync_copy(aggc, uf0_h.at[psl])
            pltpu.sync_copy(d2c, d20_h.at[psl])
            pltpu.sync_copy(hdc, hd0_h.at[psl])

        @pl.when(c == 1)
        def _():
            pltpu.sync_copy(aggc, uf1_h.at[psl])
            pltpu.sync_copy(d2c, d21_h.at[psl])
            pltpu.sync_copy(hdc, hd1_h.at[psl])

        pltpu.sync_copy(zc, agg_sh.at[psl])

    plsc.subcore_barrier()

    @pl.loop(0, _K)
    def _(k):
        pltpu.sync_copy(src_h.at[pl.ds(_eoff(0), _BLK)], sidx0)
        pltpu.sync_copy(dst2_h.at[_doff(0)], didx0)
        _gather_issue(0)
        _idx_issue(1)
        for b in range(_NBLK):
            if b >= 1:
                _scatter_drain(b - 1)
            if b + 1 < _NBLK:
                _idx_wait(b + 1)
                _gather_issue(b + 1)
            _gather_wait(b)
            _scatter_issue(b)
            if b + 2 < _NBLK:
                _idx_issue(b + 2)
        _scatter_drain(_NBLK - 1)

        plsc.subcore_barrier()
        for p in range(_RPT // _PB):
            psl = _psl(p)
            pltpu.sync_copy(agg_sh.at[psl], aggc)

            @pl.when(c == 0)
            def _():
                a = pltpu.async_copy(d20_h.at[psl], d2c, semp)
                b = pltpu.async_copy(hd0_h.at[psl], hdc, semp)
                u = pltpu.async_copy(uf0_h.at[psl], uc, semp)
                a.wait()
                b.wait()
                u.wait()

            @pl.when(c == 1)
            def _():
                a = pltpu.async_copy(d21_h.at[psl], d2c, semp)
                b = pltpu.async_copy(hd1_h.at[psl], hdc, semp)
                u = pltpu.async_copy(uf1_h.at[psl], uc, semp)
                a.wait()
                b.wait()
                u.wait()


            @pl.loop(0, _PB)
            def _(r):
                for cc in (0, 16):
                    sl = pl.ds(cc, 16)
                    aggc[r, sl] = (d2c[r, sl] * (aggc[r, sl] + uc[r, sl])
                                   + hdc[r, sl])

            @pl.when(c == 0)
            def _():
                pltpu.sync_copy(aggc, uf0_h.at[psl])

            @pl.when(c == 1)
            def _():
                pltpu.sync_copy(aggc, uf1_h.at[psl])

            pltpu.sync_copy(zc, agg_sh.at[psl])

        plsc.subcore_barrier()


def _make_prop():
    f32 = jnp.float32
    o = jax.ShapeDtypeStruct
    return pl.kernel(
        _prop_body,
        out_type=[o((_NP, 32), f32), o((_NP, 32), f32),
                  o((_NP, 32), f32), o((_NP, 32), f32),
                  o((_NP, 32), f32), o((_NP, 32), f32)],
        mesh=_sc_mesh(),
        compiler_params=pltpu.CompilerParams(use_tc_tiling_on_sc=False, needs_layout_passes=False),
        scratch_types=[
            pltpu.VMEM_SHARED((_NP, 32), f32),
            pltpu.VMEM((_BLK,), jnp.int32),
            pltpu.VMEM((_BLK,), jnp.int32),
            pltpu.VMEM((8, 128), jnp.int32),
            pltpu.VMEM((8, 128), jnp.int32),
            pltpu.VMEM((8, 128), jnp.int32),
            pltpu.VMEM((8, 128), jnp.int32),
            pltpu.VMEM((_BLK, 32), f32),
            pltpu.VMEM((_BLK, 32), f32),
            pltpu.VMEM((_PB, 32), f32),
            pltpu.VMEM((_PB, 32), f32),
            pltpu.VMEM((_PB, 32), f32),
            pltpu.VMEM((_PB, 32), f32),
            pltpu.VMEM((_PB, 32), f32),
            pltpu.SemaphoreType.DMA,
            pltpu.SemaphoreType.DMA,
            pltpu.SemaphoreType.DMA,
            pltpu.SemaphoreType.DMA,
            pltpu.SemaphoreType.DMA,
            pltpu.SemaphoreType.DMA,
        ],
    )


def _final_body(u0b, u1b, d2b, w2, b2r, lsm, xo, sm):
    r = lax.rsqrt(jnp.maximum(d2b[...], 1e-30) * (1.0 / 0.9))
    z = jnp.concatenate([u0b[...] * r, u1b[...] * r], axis=1)
    logits = jnp.dot(z, w2[...], preferred_element_type=jnp.float32) + b2r[...]
    m = jnp.max(logits, axis=1, keepdims=True)
    ex = jnp.exp(logits - m)
    ssum = jnp.sum(ex, axis=1, keepdims=True)
    xo[...] = logits[:, :_CLS]
    lsm[...] = (logits - m - jnp.log(ssum))[:, :_CLS]
    sm[...] = (ex / ssum)[:, :_CLS]


def _make_final():
    f32 = jnp.float32
    o = jax.ShapeDtypeStruct
    return pl.pallas_call(
        _final_body,
        grid=(_N // _FRB,),
        in_specs=[
            pl.BlockSpec((_FRB, 32), lambda i: (i, 0)),
            pl.BlockSpec((_FRB, 32), lambda i: (i, 0)),
            pl.BlockSpec((_FRB, 32), lambda i: (i, 0)),
            pl.BlockSpec((_H, 128), lambda i: (0, 0)),
            pl.BlockSpec((1, 128), lambda i: (0, 0)),
        ],
        out_specs=[
            pl.BlockSpec((_FRB, _CLS), lambda i: (i, 0)),
            pl.BlockSpec((_FRB, _CLS), lambda i: (i, 0)),
            pl.BlockSpec((_FRB, _CLS), lambda i: (i, 0)),
        ],
        out_shape=[o((_N, _CLS), f32), o((_N, _CLS), f32), o((_N, _CLS), f32)],
    )


def kernel(x, edge_index, e_w, idx, W1, b1, W2, b2):
    del e_w, idx
    n_extra = _EP - edge_index.shape[1]
    padv = _N + (jnp.arange(n_extra, dtype=jnp.int32) % (_NP - _N))
    src = jnp.concatenate([edge_index[0], padv])
    dst = jnp.concatenate([edge_index[1], padv])
    dst2 = dst.reshape(_DROWS, 128)

    xp = jnp.pad(x, ((0, _NP - _N), (0, 0)))
    b1r = b1.reshape(1, _H)
    w2p = jnp.pad(W2, ((0, 0), (0, 128 - _CLS)))
    b2r = jnp.concatenate(
        [b2, jnp.full((128 - _CLS,), -1e30, jnp.float32)]).reshape(1, 128)

    h0, h1 = _make_embed()(xp, W1, b1r)
    uf0, uf1, d20, _, _, _ = _make_prop()(src, dst2, h0, h1)
    lsm, xo, sm = _make_final()(uf0, uf1, d20, w2p, b2r)
    return (lsm, xo, 0.0, sm)
